# Initial kernel scaffold; baseline (speedup 1.0000x reference)
import jax
import jax.numpy as jnp
from jax.experimental import pallas as pl


def kernel(f0, f1, Wp0, bp0, Wp1, bp1, We0, be0, We1, be1, Wa, ba, qa, Wcl1, bcl1, Wcl2, bcl2, Wch1, bch1, Wch2, bch2, Wd1, bd1, Wd2, bd2, edge_index):
    raise NotImplementedError("write your pallas kernel here")


# speedup vs baseline: 7.6386x; 7.6386x over previous
import functools

import jax
import jax.numpy as jnp
from jax import lax
from jax.experimental import pallas as pl
from jax.experimental.pallas import tpu as pltpu
from jax.experimental.pallas import tpu_sc as plsc

N = 4096
D = 128
E = 262144
TAU = 0.5
K_CLU = 10

NC = 2
NS = 16
NW = NC * NS
CH = 128
EW = E // NW
NCHUNK = EW // CH
BI = 512
NI = N // BI

_f32 = jnp.float32


def _make_sc_prop(k_in: int, with_cnt: bool):
    mesh = plsc.VectorSubcoreMesh(core_axis_name="c", subcore_axis_name="s")

    out_type = [jax.ShapeDtypeStruct((k_in, 2, N, D), _f32)]
    if with_cnt:
        out_type.append(jax.ShapeDtypeStruct((2, N, 128), _f32))

    scratch = [
        pltpu.VMEM((NCHUNK, CH), jnp.int32),
        pltpu.VMEM((NCHUNK, CH), jnp.int32),
        pltpu.VMEM((CH, D), _f32),
        pltpu.VMEM((CH, D), _f32),
        pltpu.VMEM_SHARED((N, D), _f32),
        pltpu.SemaphoreType.DMA,
        pltpu.SemaphoreType.DMA,
    ]
    if with_cnt:
        scratch += [
            pltpu.VMEM((CH, 128), _f32),
            pltpu.VMEM_SHARED((N, 128), _f32),
        ]

    @functools.partial(pl.kernel, out_type=out_type, mesh=mesh,
                       scratch_types=scratch)
    def sc_prop(*refs):
        hs = refs[:k_in]
        src_hbm, dst_hbm, zeros_hbm = refs[k_in:k_in + 3]
        pos = k_in + 3
        if with_cnt:
            ones_hbm = refs[pos]; pos += 1
        out_hbm = refs[pos]; pos += 1
        if with_cnt:
            cnt_hbm = refs[pos]; pos += 1
        idx_s, idx_d, rows0, rows1, acc, sem0, sem1 = refs[pos:pos + 7]
        pos += 7
        if with_cnt:
            ones_v, cacc = refs[pos:pos + 2]

        c = lax.axis_index("c")
        s = lax.axis_index("s")
        w = s * NC + c
        rs = N // NS

        pltpu.sync_copy(src_hbm.at[pl.ds(w * NCHUNK, NCHUNK)], idx_s)
        pltpu.sync_copy(dst_hbm.at[pl.ds(w * NCHUNK, NCHUNK)], idx_d)
        if with_cnt:
            pltpu.sync_copy(ones_hbm, ones_v)

        for k in range(k_in):
            cnt_now = with_cnt and k == 0
            pltpu.sync_copy(zeros_hbm.at[pl.ds(s * rs, rs)],
                            acc.at[pl.ds(s * rs, rs)])
            if cnt_now:
                pltpu.sync_copy(zeros_hbm.at[pl.ds(s * rs, rs)],
                                cacc.at[pl.ds(s * rs, rs)])
            plsc.subcore_barrier()

            h_ref = hs[k]
            pltpu.async_copy(h_ref.at[idx_s.at[0]], rows0, sem0)

            def body(jj, carry):
                j0 = 2 * jj
                j1 = j0 + 1
                pltpu.make_async_copy(h_ref.at[idx_s.at[j0]], rows0,
                                      sem0).wait()
                pltpu.async_copy(h_ref.at[idx_s.at[j1]], rows1, sem1)
                pltpu.sync_copy(rows0, acc.at[idx_d.at[j0]], add=True)
                if cnt_now:
                    pltpu.sync_copy(ones_v, cacc.at[idx_d.at[j0]], add=True)
                pltpu.make_async_copy(h_ref.at[idx_s.at[j1]], rows1,
                                      sem1).wait()

                @pl.when(jj + 1 < NCHUNK // 2)
                def _():
                    pltpu.async_copy(h_ref.at[idx_s.at[j0 + 2]], rows0, sem0)

                pltpu.sync_copy(rows1, acc.at[idx_d.at[j1]], add=True)
                if cnt_now:
                    pltpu.sync_copy(ones_v, cacc.at[idx_d.at[j1]], add=True)
                return carry

            lax.fori_loop(0, NCHUNK // 2, body, 0)
            plsc.subcore_barrier()
            pltpu.sync_copy(acc.at[pl.ds(s * rs, rs)],
                            out_hbm.at[k, c, pl.ds(s * rs, rs)])
            if cnt_now:
                pltpu.sync_copy(cacc.at[pl.ds(s * rs, rs)],
                                cnt_hbm.at[c, pl.ds(s * rs, rs)])
            plsc.subcore_barrier()

    return sc_prop


def _sc_prop_call(h_list, src2d, dst2d, with_cnt):
    k_in = len(h_list)
    zeros = jnp.zeros((N, D), _f32)
    args = list(h_list) + [src2d, dst2d, zeros]
    if with_cnt:
        ones = jnp.zeros((CH, 128), _f32).at[:, 0].set(1.0)
        args.append(ones)
    return _make_sc_prop(k_in, with_cnt)(*args)


def _dotf(a, b):
    return jnp.dot(a, b, preferred_element_type=_f32)


def _elu(x):
    return jnp.where(x > 0, x, jnp.exp(jnp.minimum(x, 0.0)) - 1.0)


def _k_pre(f0, f1, Wp0, bp0, Wp1, bp1):
    def body(f0r, f1r, w0r, b0r, w1r, b1r, z0r, z1r):
        z0r[...] = jnp.maximum(_dotf(f0r[...], w0r[...]) + b0r[...], 0.0)
        z1r[...] = jnp.maximum(_dotf(f1r[...], w1r[...]) + b1r[...], 0.0)

    return pl.pallas_call(
        body,
        out_shape=[jax.ShapeDtypeStruct((N, D), _f32)] * 2,
    )(f0, f1, Wp0, bp0.reshape(1, D), Wp1, bp1.reshape(1, D))


def _k_layer1(z0, z1, P1, cnt, We0, be0):
    def body(z0r, z1r, P1r, cntr, Wr, br, hsr):
        deg = jnp.maximum(cntr[0, :, 0:1] + cntr[1, :, 0:1], 1.0)
        inv = 1.0 / deg
        W = Wr[...]
        b = br[...]
        for v, zr in ((0, z0r), (1, z1r)):
            ag = (P1r[v, 0] + P1r[v, 1]) * inv
            hsr[v] = jnp.maximum(_dotf(ag, W) + b, 0.0)
            hsr[2 + v] = jnp.maximum(_dotf(zr[...] - ag, W) + b, 0.0)

    return pl.pallas_call(
        body,
        out_shape=jax.ShapeDtypeStruct((4, N, D), _f32),
    )(z0, z1, P1, cnt, We0, be0.reshape(1, D))


def _k_enc2(P2, hs, cnt, We1, be1, Wa, ba, qa):
    def body(P2r, hsr, cntr, Wr, br, War, bar, qar, Zsr, zr):
        deg = jnp.maximum(cntr[0, :, 0:1] + cntr[1, :, 0:1], 1.0)
        inv = 1.0 / deg
        W = Wr[...]
        b = br[...]
        outs = []
        for k in range(4):
            ag = (P2r[k, 0] + P2r[k, 1]) * inv
            if k >= 2:
                ag = hsr[k] - ag
            o = jnp.maximum(_dotf(ag, W) + b, 0.0)
            Zsr[k] = o
            outs.append(o)
        zml = (outs[0] + outs[1]) * 0.5
        zmh = (outs[2] + outs[3]) * 0.5
        qv = qar[...]
        tl = jnp.tanh(_dotf(zml, War[...]) + bar[...])
        th = jnp.tanh(_dotf(zmh, War[...]) + bar[...])
        sl = jnp.mean(jnp.sum(tl * qv, axis=1))
        sh = jnp.mean(jnp.sum(th * qv, axis=1))
        m = jnp.maximum(sl, sh)
        el = jnp.exp(sl - m)
        eh = jnp.exp(sh - m)
        bl = el / (el + eh)
        bh = eh / (el + eh)
        zr[...] = bl * zml + bh * zmh

    return pl.pallas_call(
        body,
        out_shape=[jax.ShapeDtypeStruct((4, N, D), _f32),
                   jax.ShapeDtypeStruct((N, D), _f32)],
    )(P2, hs, cnt, We1, be1.reshape(1, D), Wa, ba.reshape(1, D),
      qa.reshape(1, D))


def _k_proj(Zs, z, Wcl1, bcl1, Wcl2, bcl2, Wch1, bch1, Wch2, bch2):
    def body(Zsr, zr, W1lr, b1lr, W2lr, b2lr, W1hr, b1hr, W2hr, b2hr, Hr):
        def proj(t, W1, b1, W2, b2):
            h = _dotf(_elu(_dotf(t, W1) + b1), W2) + b2
            nrm = jnp.sqrt(jnp.sum(h * h, axis=1, keepdims=True)) + 1e-8
            return h / nrm

        W1l, b1l, W2l, b2l = W1lr[...], b1lr[...], W2lr[...], b2lr[...]
        W1h, b1h, W2h, b2h = W1hr[...], b1hr[...], W2hr[...], b2hr[...]
        zz = zr[...]
        Hr[0] = proj(Zsr[0], W1l, b1l, W2l, b2l)
        Hr[1] = proj(Zsr[1], W1l, b1l, W2l, b2l)
        Hr[2] = proj(zz, W1l, b1l, W2l, b2l)
        Hr[3] = proj(Zsr[2], W1h, b1h, W2h, b2h)
        Hr[4] = proj(Zsr[3], W1h, b1h, W2h, b2h)
        Hr[5] = proj(zz, W1h, b1h, W2h, b2h)

    return pl.pallas_call(
        body,
        out_shape=jax.ShapeDtypeStruct((6, N, D), _f32),
    )(Zs, z, Wcl1, bcl1.reshape(1, D), Wcl2, bcl2.reshape(1, D),
      Wch1, bch1.reshape(1, D), Wch2, bch2.reshape(1, D))


def _k_sim(H):
    def body(ha_ref, hb_ref, R_ref, C_ref, Dg_ref):
        i = pl.program_id(1)
        ha = ha_ref[0]
        hb = hb_ref[0]
        logits = lax.dot_general(ha, hb, (((1,), (1,)), ((), ())),
                                 preferred_element_type=_f32) * (1.0 / TAU)
        sim = jnp.exp(logits)
        R_ref[0, 0] = jnp.sum(sim, axis=1)

        @pl.when(i == 0)
        def _():
            C_ref[...] = jnp.zeros((1, 1, N), _f32)

        C_ref[0, 0] += jnp.sum(sim, axis=0)
        hbd = hb_ref[0, pl.ds(i * BI, BI), :]
        Dg_ref[0, 0] = jnp.exp(jnp.sum(ha * hbd, axis=1) * (1.0 / TAU))

    return pl.pallas_call(
        body,
        grid=(4, NI),
        in_specs=[
            pl.BlockSpec((1, BI, D), lambda p, i: (p + p // 2, i, 0)),
            pl.BlockSpec((1, N, D), lambda p, i: (2 + 3 * (p // 2), 0, 0)),
        ],
        out_specs=[
            pl.BlockSpec((1, 1, BI), lambda p, i: (p, 0, i)),
            pl.BlockSpec((1, 1, N), lambda p, i: (p, 0, 0)),
            pl.BlockSpec((1, 1, BI), lambda p, i: (p, 0, i)),
        ],
        out_shape=[jax.ShapeDtypeStruct((4, 1, N), _f32)] * 3,
    )(H, H)


def _k_final(z, Zs, R, C, Dg, f0, f1, Wd1, bd1, Wd2, bd2):
    def body(zr, Zsr, Rr, Cr, Dgr, f0r, f1r, W1ar, W1br, b1r, W2r, b2r,
             loss_ref, lab_ref):
        W1a, W1b, b1 = W1ar[...], W1br[...], b1r[...]
        W2, b2 = W2r[...], b2r[...]
        loss_rec = 0.0
        for v, fr in ((0, f0r), (1, f1r)):
            t = _elu(_dotf(Zsr[v], W1a) + _dotf(Zsr[2 + v], W1b) + b1)
            rec = _dotf(t, W2) + b2
            f = fr[...]
            xn = rec / (jnp.sqrt(jnp.sum(rec * rec, axis=1,
                                         keepdims=True)) + 1e-8)
            yn = f / (jnp.sqrt(jnp.sum(f * f, axis=1, keepdims=True)) + 1e-8)
            cos = jnp.sum(xn * yn, axis=1)
            loss_rec = loss_rec + jnp.mean((1.0 - cos) ** 2)
        loss_rec = loss_rec * 0.5

        Rv, Cv, Dv = Rr[...], Cr[...], Dgr[...]
        l_ab = -jnp.mean(jnp.log(Dv / Rv + 1e-8), axis=1)
        l_ba = -jnp.mean(jnp.log(Dv / Cv + 1e-8), axis=1)
        loss_l = jnp.sum(0.5 * (l_ab + l_ba))

        zz = zr[...]
        cols = []
        for k in range(K_CLU):
            ck = zz[k:k + 1, :]
            d2 = jnp.sum((zz - ck) ** 2, axis=1, keepdims=True)
            cols.append(1.0 / (1.0 + d2))
        cols.append(jnp.zeros((N, 128 - K_CLU), _f32))
        q = jnp.concatenate(cols, axis=1)
        qn = q / jnp.sum(q, axis=1, keepdims=True)
        colmask = (lax.broadcasted_iota(jnp.int32, (1, 128), 1) < K_CLU)
        qcol = jnp.sum(qn, axis=0, keepdims=True)
        fq = jnp.where(colmask, qn * qn / qcol, 0.0)
        p = fq / jnp.sum(fq, axis=1, keepdims=True)
        kl = jnp.sum(p * jnp.log((p + 1e-8) / (qn + 1e-8)), axis=1)
        loss_clu = jnp.mean(kl)

        maxq = jnp.max(qn, axis=1, keepdims=True)
        ii = lax.broadcasted_iota(jnp.int32, (N, 128), 1)
        lab = jnp.min(jnp.where(qn >= maxq, ii, 128), axis=1)
        lab_ref[...] = lab.reshape(1, N)

        loss_ref[...] = jnp.reshape(loss_rec + loss_clu + loss_l, (1, 1))

    return pl.pallas_call(
        body,
        out_shape=[jax.ShapeDtypeStruct((1, 1), _f32),
                   jax.ShapeDtypeStruct((1, N), jnp.int32)],
    )(z, Zs, R, C, Dg, f0, f1, Wd1[:D], Wd1[D:], bd1.reshape(1, D),
      Wd2, bd2.reshape(1, D))


def kernel(f0, f1, Wp0, bp0, Wp1, bp1, We0, be0, We1, be1, Wa, ba, qa,
           Wcl1, bcl1, Wcl2, bcl2, Wch1, bch1, Wch2, bch2, Wd1, bd1, Wd2, bd2,
           edge_index):
    src2d = edge_index[0].reshape(E // CH, CH)
    dst2d = edge_index[1].reshape(E // CH, CH)

    z0, z1 = _k_pre(f0, f1, Wp0, bp0, Wp1, bp1)

    P1, cnt = _sc_prop_call([z0, z1], src2d, dst2d, with_cnt=True)

    hs = _k_layer1(z0, z1, P1, cnt, We0, be0)

    (P2,) = _sc_prop_call([hs[0], hs[1], hs[2], hs[3]], src2d, dst2d,
                          with_cnt=False)

    Zs, z = _k_enc2(P2, hs, cnt, We1, be1, Wa, ba, qa)

    H = _k_proj(Zs, z, Wcl1, bcl1, Wcl2, bcl2, Wch1, bch1, Wch2, bch2)

    R, C, Dg = _k_sim(H)
    R, C, Dg = R.reshape(4, N), C.reshape(4, N), Dg.reshape(4, N)

    loss2d, lab2d = _k_final(z, Zs, R, C, Dg, f0, f1, Wd1, bd1, Wd2, bd2)

    return loss2d.reshape(()), lab2d.reshape(N)
